# baseline (speedup 1.0000x reference)
import functools

import jax
import jax.numpy as jnp
from jax import lax
from jax.experimental import pallas as pl
from jax.experimental.pallas import tpu as pltpu
from jax.experimental.pallas import tpu_sc as plsc

N_NODES = 10000
N_EDGES = 320000
D_IN = 128

NC = 2
NS = 16
NW = NC * NS
CHUNK = 128
N_PAD = 10112
RPT = N_PAD // NS
EPT_CHUNKS = 84
E_PAD = NW * EPT_CHUNKS * CHUNK


def _sc_scatter(hw, zeros, src_r, dst_r, d, fsplit, nb):
    n_chunks = src_r.shape[1]
    assert n_chunks % nb == 0 and nb >= 2
    mesh = plsc.VectorSubcoreMesh(core_axis_name="c", subcore_axis_name="s")

    @functools.partial(
        pl.kernel,
        out_type=jax.ShapeDtypeStruct((NC, N_PAD, d), jnp.float32),
        mesh=mesh,
        scratch_types=[
            pltpu.VMEM((n_chunks, CHUNK), jnp.int32),
            pltpu.VMEM((n_chunks, CHUNK), jnp.int32),
            pltpu.VMEM((nb, CHUNK, d), jnp.float32),
            pltpu.VMEM_SHARED((N_PAD, d), jnp.float32),
        ] + [pltpu.SemaphoreType.DMA] * nb,
        compiler_params=pltpu.CompilerParams(use_tc_tiling_on_sc=False),
    )
    def k(hw_hbm, z_hbm, src_hbm, dst_hbm, out_hbm,
          src_v, dst_v, rows_v, acc_sh, *gsems):
        c = lax.axis_index("c")
        s = lax.axis_index("s")
        table = hw_hbm.at[c] if fsplit else hw_hbm
        wid = s if fsplit else s * NC + c
        cp_src = pltpu.async_copy(src_hbm.at[wid], src_v, gsems[0])
        cp_dst = pltpu.async_copy(dst_hbm.at[wid], dst_v, gsems[1])
        pltpu.sync_copy(z_hbm.at[pl.ds(s * RPT, RPT)],
                        acc_sh.at[pl.ds(s * RPT, RPT)])
        cp_src.wait()
        cp_dst.wait()
        plsc.subcore_barrier()

        def gissue(j, b):
            pltpu.async_copy(table.at[src_v.at[j]], rows_v.at[b], gsems[b])

        def gwait(j, b):
            pltpu.make_async_copy(table.at[src_v.at[j]], rows_v.at[b],
                                  gsems[b]).wait()

        def ssync(j, b):
            pltpu.sync_copy(rows_v.at[b], acc_sh.at[dst_v.at[j]], add=True)

        for jj in range(nb - 1):
            gissue(jj, jj)

        @pl.loop(0, n_chunks - nb, step=nb)
        def _(j0):
            for b in range(nb):
                j = j0 + b
                gissue(j + nb - 1, (b + nb - 1) % nb)
                gwait(j, b)
                ssync(j, b)

        for jj in range(n_chunks - nb, n_chunks):
            if jj + nb - 1 < n_chunks:
                gissue(jj + nb - 1, (jj + nb - 1) % nb)
            gwait(jj, jj % nb)
            ssync(jj, jj % nb)

        plsc.subcore_barrier()
        pltpu.sync_copy(acc_sh.at[pl.ds(s * RPT, RPT)],
                        out_hbm.at[c].at[pl.ds(s * RPT, RPT)])

    return k(hw, zeros, src_r, dst_r)


def _tc_scale(h, wcol, split):
    def body(h_ref, wc_ref, o_ref):
        hw = wc_ref[...] * h_ref[...]
        if split:
            dh = hw.shape[1] // 2
            o_ref[0, pl.ds(0, N_NODES), :] = hw[:, :dh]
            o_ref[1, pl.ds(0, N_NODES), :] = hw[:, dh:]
        else:
            o_ref[pl.ds(0, N_NODES), :] = hw

    dim = h.shape[1]
    shp = (2, N_PAD, dim // 2) if split else (N_PAD, dim)
    return pl.pallas_call(
        body,
        out_shape=jax.ShapeDtypeStruct(shp, jnp.float32),
    )(h, wcol)


def _tc_layer(p, h, w_mat, b, wcol, concat, act):
    def body(p_ref, h_ref, w_ref, b_ref, wc_ref, hn_ref, hwn_ref):
        hr = h_ref[pl.ds(0, N_NODES), :]
        if concat:
            x = jnp.concatenate(
                [p_ref[0, pl.ds(0, N_NODES), :],
                 p_ref[1, pl.ds(0, N_NODES), :]], axis=1) + hr
        else:
            x = (p_ref[0, pl.ds(0, N_NODES), :]
                 + p_ref[1, pl.ds(0, N_NODES), :] + hr)
        y = jnp.dot(x, w_ref[...], preferred_element_type=jnp.float32) + b_ref[...]
        y = jnp.maximum(y, 0.0) if act == "relu" else jax.nn.sigmoid(y)
        hn_ref[pl.ds(0, N_NODES), :] = y
        hwn_ref[pl.ds(0, N_NODES), :] = wc_ref[...] * y

    d = w_mat.shape[1]
    return pl.pallas_call(
        body,
        out_shape=[
            jax.ShapeDtypeStruct((N_PAD, d), jnp.float32),
            jax.ShapeDtypeStruct((N_PAD, d), jnp.float32),
        ],
    )(p, h, w_mat, b, wcol)


def _tc_last(p, h, w_mat, b):
    def body(p_ref, h_ref, w_ref, b_ref, o_ref):
        x = (p_ref[0, pl.ds(0, N_NODES), :]
             + p_ref[1, pl.ds(0, N_NODES), :]
             + h_ref[pl.ds(0, N_NODES), :])
        o_ref[...] = jax.nn.sigmoid(
            jnp.dot(x, w_ref[...], preferred_element_type=jnp.float32)
            + b_ref[...])

    d = w_mat.shape[1]
    return pl.pallas_call(
        body,
        out_shape=jax.ShapeDtypeStruct((N_NODES, d), jnp.float32),
    )(p, h, w_mat, b)


def kernel(structure, H, input_weight, W1, b1, W2, b2, W3, b3):
    src = structure[0]
    dst = structure[1]
    pad = E_PAD - N_EDGES
    fill = N_NODES + (jnp.arange(pad, dtype=jnp.int32) % (N_PAD - N_NODES))
    src_flat = jnp.concatenate([src, fill])
    dst_flat = jnp.concatenate([dst, fill])
    src_r1 = src_flat.reshape(NS, 2 * EPT_CHUNKS, CHUNK)
    dst_r1 = dst_flat.reshape(NS, 2 * EPT_CHUNKS, CHUNK)
    src_r = src_flat.reshape(NW, EPT_CHUNKS, CHUNK)
    dst_r = dst_flat.reshape(NW, EPT_CHUNKS, CHUNK)

    wcol = input_weight.reshape(N_NODES, 1)
    z = jnp.zeros((N_PAD, D_IN), jnp.float32)

    hw1 = _tc_scale(H, wcol, True)
    p1 = _sc_scatter(hw1, z[:, :64], src_r1, dst_r1, 64, True, 4)
    h1, hw2 = _tc_layer(p1, H, W1, b1.reshape(1, -1), wcol, True, "relu")
    p2 = _sc_scatter(hw2, z[:, :32], src_r, dst_r, 32, False, 6)
    h2, hw3 = _tc_layer(p2, h1, W2, b2.reshape(1, -1), wcol, False, "relu")
    p3 = _sc_scatter(hw3, z[:, :16], src_r, dst_r, 16, False, 6)
    return _tc_last(p3, h2, W3, b3.reshape(1, -1))
